# Initial kernel scaffold; baseline (speedup 1.0000x reference)
import functools

import jax
import jax.numpy as jnp
from jax import lax
from jax.experimental import pallas as pl
from jax.experimental.pallas import tpu as pltpu
from jax.experimental.pallas import tpu_sc as plsc

N, E, F = 10000, 320000, 128

NC, NS = 2, 16
NW = NC * NS

C = 128
KW = -(-E // (NW * C))
E_PAD = NW * KW * C
RPT = 626
N_PAD = NS * RPT

_MESH = plsc.VectorSubcoreMesh(
    core_axis_name="c", subcore_axis_name="s", num_cores=NC, num_subcores=NS
)


@functools.partial(
    pl.kernel,
    out_type=jax.ShapeDtypeStruct((NC, N_PAD, F), jnp.float32),
    mesh=_MESH,
    scratch_types=[
        pltpu.VMEM((KW, C), jnp.int32),
        pltpu.VMEM((KW, C), jnp.int32),
        pltpu.VMEM((C, F), jnp.float32),
        pltpu.VMEM_SHARED((N_PAD, F), jnp.float32),
        pltpu.SemaphoreType.DMA,
    ],
)
def _sc_segment_sum(h, srcs, dsts, zinit, out, src_v, dst_v, rows_v, acc, sem):
    cid = lax.axis_index("c")
    sid = lax.axis_index("s")
    wid = sid * NC + cid

    pltpu.sync_copy(zinit, acc.at[pl.ds(sid * RPT, RPT)])
    pltpu.sync_copy(srcs.at[wid], src_v)
    pltpu.sync_copy(dsts.at[wid], dst_v)
    plsc.subcore_barrier()

    def step(g, carry):
        pltpu.async_copy(h.at[src_v.at[g]], rows_v, sem).wait()
        pltpu.sync_copy(rows_v, acc.at[dst_v.at[g]], add=True)
        return carry

    lax.fori_loop(0, KW, step, 0)
    plsc.subcore_barrier()
    pltpu.sync_copy(
        acc.at[pl.ds(sid * RPT, RPT)], out.at[cid, pl.ds(sid * RPT, RPT)]
    )


BN = 1000


def _tc_body(last, x_ref, a0_ref, a1_ref, w_ref, b_ref, eps_ref, o_ref):
    y = (1.0 + eps_ref[0, 0]) * x_ref[...] + a0_ref[0] + a1_ref[0]
    z = jnp.dot(y, w_ref[...], preferred_element_type=jnp.float32) + b_ref[...]
    z = jnp.maximum(z, 0.0)
    if last:
        m = jnp.max(z, axis=1, keepdims=True)
        z = z - m - jnp.log(jnp.sum(jnp.exp(z - m), axis=1, keepdims=True))
    o_ref[...] = z


def _tc_layer(x, agg, w, b, eps, last):
    return pl.pallas_call(
        functools.partial(_tc_body, last),
        grid=(N // BN,),
        in_specs=[
            pl.BlockSpec((BN, F), lambda i: (i, 0)),
            pl.BlockSpec((1, BN, F), lambda i: (0, i, 0)),
            pl.BlockSpec((1, BN, F), lambda i: (1, i, 0)),
            pl.BlockSpec((F, F), lambda i: (0, 0)),
            pl.BlockSpec((1, F), lambda i: (0, 0)),
            pl.BlockSpec(memory_space=pltpu.SMEM),
        ],
        out_specs=pl.BlockSpec((BN, F), lambda i: (i, 0)),
        out_shape=jax.ShapeDtypeStruct((N, F), jnp.float32),
    )(x, agg, agg, w, b, eps.reshape(1, 1))


def kernel(x, edge_index, W1, b1, W2, b2, W3, b3, eps1, eps2, eps3):
    src = edge_index[0].astype(jnp.int32)
    dst = edge_index[1].astype(jnp.int32)
    pad = E_PAD - E
    srcs = jnp.concatenate([src, jnp.zeros((pad,), jnp.int32)]).reshape(NW, KW, C)
    dsts = jnp.concatenate([dst, jnp.full((pad,), N, jnp.int32)]).reshape(NW, KW, C)
    zinit = jnp.zeros((RPT, F), jnp.float32)

    h = x
    for w, b, eps, last in (
        (W1, b1, eps1, False),
        (W2, b2, eps2, False),
        (W3, b3, eps3, True),
    ):
        agg = _sc_segment_sum(h, srcs, dsts, zinit)
        h = _tc_layer(h, agg, w, b, eps, last)
    return h


# speedup vs baseline: 4.1017x; 4.1017x over previous
import functools

import jax
import jax.numpy as jnp
from jax import lax
from jax.experimental import pallas as pl
from jax.experimental.pallas import tpu as pltpu
from jax.experimental.pallas import tpu_sc as plsc

N, E, F = 10000, 320000, 128

NC, NS = 2, 16
NW = NC * NS

C = 128
KW = -(-E // (NW * C))
E_PAD = NW * KW * C
RPT = 632
N_PAD = NS * RPT

_MESH = plsc.VectorSubcoreMesh(
    core_axis_name="c", subcore_axis_name="s", num_cores=NC, num_subcores=NS
)


@functools.partial(
    pl.kernel,
    out_type=jax.ShapeDtypeStruct((NC, N_PAD, F), jnp.float32),
    mesh=_MESH,
    scratch_types=[
        pltpu.VMEM((KW, C), jnp.int32),
        pltpu.VMEM((KW, C), jnp.int32),
        pltpu.VMEM((C, F), jnp.float32),
        pltpu.VMEM_SHARED((N_PAD, F), jnp.float32),
        pltpu.SemaphoreType.DMA,
    ],
)
def _sc_segment_sum(h, srcs, dsts, zinit, out, src_v, dst_v, rows_v, acc, sem):
    cid = lax.axis_index("c")
    sid = lax.axis_index("s")
    wid = sid * NC + cid

    pltpu.sync_copy(zinit, acc.at[pl.ds(sid * RPT, RPT)])
    pltpu.sync_copy(srcs.at[wid], src_v)
    pltpu.sync_copy(dsts.at[wid], dst_v)
    plsc.subcore_barrier()

    def step(g, carry):
        pltpu.async_copy(h.at[src_v.at[g]], rows_v, sem).wait()
        pltpu.sync_copy(rows_v, acc.at[dst_v.at[g]], add=True)
        return carry

    lax.fori_loop(0, KW, step, 0)
    plsc.subcore_barrier()
    pltpu.sync_copy(
        acc.at[pl.ds(sid * RPT, RPT)], out.at[cid, pl.ds(sid * RPT, RPT)]
    )


BN = 1000


def _tc_body(last, x_ref, a0_ref, a1_ref, w_ref, b_ref, eps_ref, o_ref):
    y = (1.0 + eps_ref[0, 0]) * x_ref[...] + a0_ref[0] + a1_ref[0]
    z = jnp.dot(y, w_ref[...], preferred_element_type=jnp.float32) + b_ref[...]
    z = jnp.maximum(z, 0.0)
    if last:
        m = jnp.max(z, axis=1, keepdims=True)
        z = z - m - jnp.log(jnp.sum(jnp.exp(z - m), axis=1, keepdims=True))
    o_ref[...] = z


def _tc_layer(x, agg, w, b, eps, last):
    return pl.pallas_call(
        functools.partial(_tc_body, last),
        grid=(N // BN,),
        in_specs=[
            pl.BlockSpec((BN, F), lambda i: (i, 0)),
            pl.BlockSpec((1, BN, F), lambda i: (0, i, 0)),
            pl.BlockSpec((1, BN, F), lambda i: (1, i, 0)),
            pl.BlockSpec((F, F), lambda i: (0, 0)),
            pl.BlockSpec((1, F), lambda i: (0, 0)),
            pl.BlockSpec(memory_space=pltpu.SMEM),
        ],
        out_specs=pl.BlockSpec((BN, F), lambda i: (i, 0)),
        out_shape=jax.ShapeDtypeStruct((N, F), jnp.float32),
    )(x, agg, agg, w, b.reshape(1, F), eps.reshape(1, 1))


def kernel(x, edge_index, W1, b1, W2, b2, W3, b3, eps1, eps2, eps3):
    src = edge_index[0].astype(jnp.int32)
    dst = edge_index[1].astype(jnp.int32)
    pad = E_PAD - E
    srcs = jnp.concatenate([src, jnp.zeros((pad,), jnp.int32)]).reshape(NW, KW, C)
    dsts = jnp.concatenate([dst, jnp.full((pad,), N, jnp.int32)]).reshape(NW, KW, C)
    zinit = jnp.zeros((RPT, F), jnp.float32)

    h = x
    for w, b, eps, last in (
        (W1, b1, eps1, False),
        (W2, b2, eps2, False),
        (W3, b3, eps3, True),
    ):
        agg = _sc_segment_sum(h, srcs, dsts, zinit)
        h = _tc_layer(h, agg, w, b, eps, last)
    return h


# speedup vs baseline: 4.5580x; 1.1112x over previous
import functools

import jax
import jax.numpy as jnp
from jax import lax
from jax.experimental import pallas as pl
from jax.experimental.pallas import tpu as pltpu
from jax.experimental.pallas import tpu_sc as plsc

N, E, F = 10000, 320000, 128

NC, NS = 2, 16
NW = NC * NS

C = 128
KW = -(-E // (NW * C))
E_PAD = NW * KW * C
RPT = 632
N_PAD = NS * RPT

_MESH = plsc.VectorSubcoreMesh(
    core_axis_name="c", subcore_axis_name="s", num_cores=NC, num_subcores=NS
)


@functools.partial(
    pl.kernel,
    out_type=jax.ShapeDtypeStruct((NC, N_PAD, F), jnp.float32),
    mesh=_MESH,
    scratch_types=[
        pltpu.VMEM((2, C), jnp.int32),
        pltpu.VMEM((2, C), jnp.int32),
        pltpu.VMEM((C, F), jnp.float32),
        pltpu.VMEM((C, F), jnp.float32),
        pltpu.VMEM_SHARED((N_PAD, F), jnp.float32),
        pltpu.SemaphoreType.DMA,
        pltpu.SemaphoreType.DMA,
    ],
)
def _sc_segment_sum(
    h, idx, zinit, out, ib0, ib1, rows0, rows1, acc, sem0, sem1
):
    cid = lax.axis_index("c")
    sid = lax.axis_index("s")
    wid = sid * NC + cid

    pltpu.sync_copy(zinit, acc.at[pl.ds(sid * RPT, RPT)])
    pltpu.sync_copy(idx.at[wid, 0], ib0)
    pltpu.sync_copy(idx.at[wid, 1], ib1)
    plsc.subcore_barrier()

    pltpu.async_copy(h.at[ib0.at[0]], rows0, sem0)

    def step(p, carry):
        g0 = 2 * p
        pltpu.make_async_copy(h.at[ib0.at[0]], rows0, sem0).wait()
        pltpu.async_copy(h.at[ib1.at[0]], rows1, sem1)
        pltpu.sync_copy(rows0, acc.at[ib0.at[1]], add=True)
        pltpu.sync_copy(idx.at[wid, g0 + 2], ib0)
        pltpu.make_async_copy(h.at[ib1.at[0]], rows1, sem1).wait()
        pltpu.async_copy(h.at[ib0.at[0]], rows0, sem0)
        pltpu.sync_copy(rows1, acc.at[ib1.at[1]], add=True)
        pltpu.sync_copy(idx.at[wid, g0 + 3], ib1)
        return carry

    lax.fori_loop(0, (KW - 1) // 2, step, 0)
    pltpu.make_async_copy(h.at[ib0.at[0]], rows0, sem0).wait()
    pltpu.sync_copy(rows0, acc.at[ib0.at[1]], add=True)
    plsc.subcore_barrier()
    pltpu.sync_copy(
        acc.at[pl.ds(sid * RPT, RPT)], out.at[cid, pl.ds(sid * RPT, RPT)]
    )


BN = 1000


def _tc_body(last, x_ref, a0_ref, a1_ref, w_ref, b_ref, eps_ref, o_ref):
    y = (1.0 + eps_ref[0, 0]) * x_ref[...] + a0_ref[0] + a1_ref[0]
    z = jnp.dot(y, w_ref[...], preferred_element_type=jnp.float32) + b_ref[...]
    z = jnp.maximum(z, 0.0)
    if last:
        m = jnp.max(z, axis=1, keepdims=True)
        z = z - m - jnp.log(jnp.sum(jnp.exp(z - m), axis=1, keepdims=True))
    o_ref[...] = z


def _tc_layer(x, agg, w, b, eps, last):
    return pl.pallas_call(
        functools.partial(_tc_body, last),
        grid=(N // BN,),
        in_specs=[
            pl.BlockSpec((BN, F), lambda i: (i, 0)),
            pl.BlockSpec((1, BN, F), lambda i: (0, i, 0)),
            pl.BlockSpec((1, BN, F), lambda i: (1, i, 0)),
            pl.BlockSpec((F, F), lambda i: (0, 0)),
            pl.BlockSpec((1, F), lambda i: (0, 0)),
            pl.BlockSpec(memory_space=pltpu.SMEM),
        ],
        out_specs=pl.BlockSpec((BN, F), lambda i: (i, 0)),
        out_shape=jax.ShapeDtypeStruct((N, F), jnp.float32),
    )(x, agg, agg, w, b.reshape(1, F), eps.reshape(1, 1))


def kernel(x, edge_index, W1, b1, W2, b2, W3, b3, eps1, eps2, eps3):
    src = edge_index[0].astype(jnp.int32)
    dst = edge_index[1].astype(jnp.int32)
    pad = E_PAD - E
    srcs = jnp.concatenate([src, jnp.zeros((pad,), jnp.int32)]).reshape(NW, KW, C)
    dsts = jnp.concatenate([dst, jnp.full((pad,), N, jnp.int32)]).reshape(NW, KW, C)
    idx = jnp.stack([srcs, dsts], axis=2)
    idx = jnp.concatenate([idx, jnp.zeros((NW, 1, 2, C), jnp.int32)], axis=1)
    zinit = jnp.zeros((RPT, F), jnp.float32)

    h = x
    for w, b, eps, last in (
        (W1, b1, eps1, False),
        (W2, b2, eps2, False),
        (W3, b3, eps3, True),
    ):
        agg = _sc_segment_sum(h, idx, zinit)
        h = _tc_layer(h, agg, w, b, eps, last)
    return h
